# baseline (speedup 1.0000x reference)
import functools

import jax
import jax.numpy as jnp
from jax import lax
from jax.experimental import pallas as pl
from jax.experimental.pallas import tpu as pltpu
from jax.experimental.pallas import tpu_sc as plsc


S, M, E = 2048, 4096, 16
C = 2 * S // E
NB = 8
TB = S // NB
NC, NS, L = 2, 16, 16
NW = NC * NS
TPW = S // NW
G = TPW // L
PAD = 128
F32, I32 = jnp.float32, jnp.int32


def _matmul_body(w_ref, x_ref, out_ref):
    out_ref[...] = jax.lax.dot_general(
        w_ref[...], x_ref[...],
        dimension_numbers=(((1,), (1,)), ((), ())),
        preferred_element_type=F32,
    ).reshape(1, E, TB)


def _logits_tc_T(x, W):
    return pl.pallas_call(
        _matmul_body,
        grid=(NB,),
        in_specs=[
            pl.BlockSpec((E, M), lambda i: (0, 0)),
            pl.BlockSpec((TB, M), lambda i: (i, 0)),
        ],
        out_specs=pl.BlockSpec((1, E, TB), lambda i: (i, 0, 0)),
        out_shape=jax.ShapeDtypeStruct((NB, E, TB), F32),
    )(W, x)


_mesh = plsc.VectorSubcoreMesh(core_axis_name="c", subcore_axis_name="s")


def _tree_reduce(op, xs):
    xs = list(xs)
    while len(xs) > 1:
        xs = [op(xs[i], xs[i + 1]) for i in range(0, len(xs) - 1, 2)] + (
            [xs[-1]] if len(xs) % 2 else [])
    return xs[0]


@functools.partial(
    pl.kernel,
    compiler_params=pltpu.CompilerParams(needs_layout_passes=False),
    out_type=(
        jax.ShapeDtypeStruct((S,), I32),
        jax.ShapeDtypeStruct((S,), I32),
        jax.ShapeDtypeStruct((S,), I32),
        jax.ShapeDtypeStruct((S,), I32),
        jax.ShapeDtypeStruct((S,), F32),
        jax.ShapeDtypeStruct((S,), F32),
        jax.ShapeDtypeStruct((NW * PAD,), I32),
        jax.ShapeDtypeStruct((NW * PAD,), I32),
        jax.ShapeDtypeStruct((NW * PAD,), F32),
    ),
    mesh=_mesh,
    scratch_types=[
        pltpu.VMEM((E * TPW,), F32),
        pltpu.VMEM((TPW,), I32),
        pltpu.VMEM((TPW,), I32),
        pltpu.VMEM((TPW,), I32),
        pltpu.VMEM((TPW,), I32),
        pltpu.VMEM((TPW,), F32),
        pltpu.VMEM((TPW,), F32),
        pltpu.VMEM((PAD,), I32),
        pltpu.VMEM((PAD,), I32),
        pltpu.VMEM((PAD,), F32),
        pltpu.SemaphoreType.DMA,
    ],
)
def _gate_a(lt_hbm, e1_hbm, e2_hbm, l1_hbm, l2_hbm, g1_hbm, g2_hbm,
            c1_hbm, c2_hbm, gs_hbm,
            lt_v, e1_v, e2_v, l1_v, l2_v, g1_v, g2_v, c1_v, c2_v, gs_v, sem):
    wid = lax.axis_index("s") * NC + lax.axis_index("c")
    base = wid * TPW
    blk = wid // (TB // TPW)
    tok0 = (wid % (TB // TPW)) * TPW
    in_cps = [
        pltpu.async_copy(lt_hbm.at[blk, e, pl.ds(tok0, TPW)],
                         lt_v.at[pl.ds(e * TPW, TPW)], sem)
        for e in range(E)
    ]
    zi = jnp.zeros((L,), I32)
    zf = jnp.zeros((L,), F32)
    for p in range(PAD // L):
        c1_v[pl.ds(p * L, L)] = zi
        c2_v[pl.ds(p * L, L)] = zi
        gs_v[pl.ds(p * L, L)] = zf
    for cp in in_cps:
        cp.wait()

    lanes = lax.iota(I32, L)
    neg_inf = jnp.full((L,), -jnp.inf, F32)
    sentinel = jnp.full((L,), E, I32)

    cnt1 = zi
    cnt2 = zi
    acc = [zf] * E
    for g in range(G):
        off = g * L
        le = [lt_v[pl.ds(e * TPW + off, L)] for e in range(E)]
        mx = _tree_reduce(jnp.maximum, le)
        ex = [jnp.exp(le[e] - mx) for e in range(E)]
        ssum = _tree_reduce(jnp.add, ex)
        gates = [ex[e] / ssum for e in range(E)]
        mg = _tree_reduce(jnp.maximum, gates)
        e1 = sentinel
        for e in range(E):
            e1 = jnp.where((gates[e] == mg) & (e1 == E), e, e1)
        g1 = mg
        ml = [jnp.where(e1 == e, neg_inf, le[e]) for e in range(E)]
        m2 = _tree_reduce(jnp.maximum, ml)
        e2 = sentinel
        g2 = zf
        for e in range(E):
            is2 = (ml[e] == m2) & (e2 == E)
            g2 = jnp.where(is2, gates[e], g2)
            e2 = jnp.where(is2, e, e2)
        c1_v[pl.ds(0, L)] = cnt1
        c2_v[pl.ds(0, L)] = cnt2
        prior1 = plsc.load_gather(c1_v.at[pl.ds(0, L)], [e1])
        prior2 = plsc.load_gather(c2_v.at[pl.ds(0, L)], [e2])
        rank1 = zi
        rank2 = zi
        for e in range(E):
            m1e = e1 == e
            rank1 = jnp.where(m1e, plsc.cumsum(m1e.astype(I32)) - 1, rank1)
            cnt1 = cnt1 + jnp.where(lanes == e,
                                    plsc.all_reduce_population_count(m1e), 0)
            m2e = e2 == e
            rank2 = jnp.where(m2e, plsc.cumsum(m2e.astype(I32)) - 1, rank2)
            cnt2 = cnt2 + jnp.where(lanes == e,
                                    plsc.all_reduce_population_count(m2e), 0)
            acc[e] = acc[e] + gates[e]
        e1_v[pl.ds(off, L)] = e1
        e2_v[pl.ds(off, L)] = e2
        l1_v[pl.ds(off, L)] = prior1 + rank1
        l2_v[pl.ds(off, L)] = prior2 + rank2
        g1_v[pl.ds(off, L)] = g1
        g2_v[pl.ds(off, L)] = g2

    c1_v[pl.ds(0, L)] = cnt1
    c2_v[pl.ds(0, L)] = cnt2
    gsr = zf
    for e in range(E):
        gsr = gsr + jnp.where(lanes == e, jnp.sum(acc[e]), 0.0)
    gs_v[pl.ds(0, L)] = gsr

    cps = [
        pltpu.async_copy(e1_v, e1_hbm.at[pl.ds(base, TPW)], sem),
        pltpu.async_copy(e2_v, e2_hbm.at[pl.ds(base, TPW)], sem),
        pltpu.async_copy(l1_v, l1_hbm.at[pl.ds(base, TPW)], sem),
        pltpu.async_copy(l2_v, l2_hbm.at[pl.ds(base, TPW)], sem),
        pltpu.async_copy(g1_v, g1_hbm.at[pl.ds(base, TPW)], sem),
        pltpu.async_copy(g2_v, g2_hbm.at[pl.ds(base, TPW)], sem),
        pltpu.async_copy(c1_v, c1_hbm.at[pl.ds(wid * PAD, PAD)], sem),
        pltpu.async_copy(c2_v, c2_hbm.at[pl.ds(wid * PAD, PAD)], sem),
        pltpu.async_copy(gs_v, gs_hbm.at[pl.ds(wid * PAD, PAD)], sem),
    ]
    for cp in cps:
        cp.wait()


def _epilogue_body(e1_ref, e2_ref, l1_ref, l2_ref, g1_ref, g2_ref,
                   c1_ref, c2_ref, gs_ref, comb_ref, disp_ref, laux_ref):
    i = pl.program_id(0)
    wpb = TB // TPW

    ident = (lax.broadcasted_iota(I32, (TB, TB), 0)
             == lax.broadcasted_iota(I32, (TB, TB), 1)).astype(F32)
    stacked = jnp.concatenate(
        [e1_ref[pl.ds(i, 1), :].astype(F32),
         e2_ref[pl.ds(i, 1), :].astype(F32),
         l1_ref[pl.ds(i, 1), :].astype(F32),
         l2_ref[pl.ds(i, 1), :].astype(F32),
         g1_ref[pl.ds(i, 1), :],
         g2_ref[pl.ds(i, 1), :]], axis=0)
    cols = jax.lax.dot_general(ident, stacked,
                               dimension_numbers=(((1,), (1,)), ((), ())),
                               preferred_element_type=F32,
                               precision=jax.lax.Precision.HIGHEST)
    e1c = cols[:, 0:1]
    e2c = cols[:, 1:2]
    l1c = cols[:, 2:3]
    l2c = cols[:, 3:4]
    g1c = cols[:, 4:5]
    g2c = cols[:, 5:6]

    tri = (lax.broadcasted_iota(I32, (NW, NW), 0)
           > lax.broadcasted_iota(I32, (NW, NW), 1)).astype(F32)
    c1f = c1_ref[...].astype(F32)
    c2f = c2_ref[...].astype(F32)
    pre1 = jax.lax.dot_general(tri, c1f, (((1,), (0,)), ((), ())),
                               preferred_element_type=F32)
    pre2 = jax.lax.dot_general(tri, c2f, (((1,), (0,)), ((), ())),
                               preferred_element_type=F32)
    tot1 = jnp.sum(c1f, axis=0, keepdims=True)
    off2 = pre2 + tot1

    rsel = (lax.broadcasted_iota(I32, (TB, NW), 0) // TPW + i * wpb
            == lax.broadcasted_iota(I32, (TB, NW), 1)).astype(F32)
    pre1tok = jax.lax.dot_general(rsel, pre1, (((1,), (0,)), ((), ())),
                                  preferred_element_type=F32)
    off2tok = jax.lax.dot_general(rsel, off2, (((1,), (0,)), ((), ())),
                                  preferred_element_type=F32)

    elane = lax.broadcasted_iota(I32, (TB, PAD), 1).astype(F32)
    oh1 = (elane == e1c).astype(F32)
    oh2 = (elane == e2c).astype(F32)
    loc1 = l1c + jnp.sum(oh1 * pre1tok, axis=1, keepdims=True)
    loc2 = l2c + jnp.sum(oh2 * off2tok, axis=1, keepdims=True)

    g1k = jnp.where(loc1 < C, g1c, 0.0)
    g2k = jnp.where(loc2 < C, g2c, 0.0)
    denom = jnp.maximum(g1k + g2k, jnp.float32(jnp.finfo(jnp.float32).eps))
    g1n = g1k / denom
    g2n = g2k / denom

    clane = lax.broadcasted_iota(I32, (TB, C), 1).astype(F32)
    combine = (g1n * (clane == loc1).astype(F32)
               + g2n * (clane == loc2).astype(F32))
    comb_ref[...] = combine
    disp_ref[...] = combine != 0.0

    @pl.when(i == 0)
    def _():
        gtot = jnp.sum(gs_ref[...], axis=0, keepdims=True)
        la = jnp.sum(gtot * tot1) * (1.0 / (E * S * S))
        laux_ref[...] = jnp.full((1, 1), 1.0, F32) * la


def _epilogue_tc(e1, e2, l1, l2, g1, g2, c1, c2, gs):
    grid8 = pl.BlockSpec((NB, TB), lambda i: (0, 0))
    cnt_spec = pl.BlockSpec((NW, PAD), lambda i: (0, 0))
    return pl.pallas_call(
        _epilogue_body,
        grid=(NB,),
        in_specs=[grid8, grid8, grid8, grid8, grid8, grid8,
                  cnt_spec, cnt_spec, cnt_spec],
        out_specs=(
            pl.BlockSpec((TB, C), lambda i: (i, 0)),
            pl.BlockSpec((TB, C), lambda i: (i, 0)),
            pl.BlockSpec((1, 1), lambda i: (0, 0)),
        ),
        out_shape=(
            jax.ShapeDtypeStruct((S, C), F32),
            jax.ShapeDtypeStruct((S, C), jnp.bool_),
            jax.ShapeDtypeStruct((1, 1), F32),
        ),
    )(e1.reshape(NB, TB), e2.reshape(NB, TB), l1.reshape(NB, TB),
      l2.reshape(NB, TB), g1.reshape(NB, TB), g2.reshape(NB, TB),
      c1.reshape(NW, PAD), c2.reshape(NW, PAD), gs.reshape(NW, PAD))


def kernel(input, W):
    lt = _logits_tc_T(input, W)
    e1, e2, l1, l2, g1, g2, c1, c2, gs = _gate_a(lt)
    comb, disp, laux = _epilogue_tc(e1, e2, l1, l2, g1, g2, c1, c2, gs)
    return (laux.reshape(()), comb.reshape(S, 1, C), disp.reshape(S, 1, C))


# speedup vs baseline: 1.3335x; 1.3335x over previous
import functools

import jax
import jax.numpy as jnp
from jax import lax
from jax.experimental import pallas as pl
from jax.experimental.pallas import tpu as pltpu
from jax.experimental.pallas import tpu_sc as plsc


S, M, E = 2048, 4096, 16
C = 2 * S // E
NC, NS, L = 2, 16, 16
NW = NC * NS
TPW = S // NW
G = TPW // L
F32, I32 = jnp.float32, jnp.int32


def _matmul_body(w_ref, x_ref, out_ref):
    out_ref[...] = jax.lax.dot_general(
        w_ref[...], x_ref[...],
        dimension_numbers=(((1,), (1,)), ((), ())),
        preferred_element_type=F32,
    ).reshape(1, E, S // 8)


def _logits_tc_T(x, W):
    return pl.pallas_call(
        _matmul_body,
        grid=(8,),
        in_specs=[
            pl.BlockSpec((E, M), lambda i: (0, 0)),
            pl.BlockSpec((S // 8, M), lambda i: (i, 0)),
        ],
        out_specs=pl.BlockSpec((1, E, S // 8), lambda i: (i, 0, 0)),
        out_shape=jax.ShapeDtypeStruct((8, E, S // 8), F32),
    )(W, x)


_mesh = plsc.VectorSubcoreMesh(core_axis_name="c", subcore_axis_name="s")


def _tree_reduce(op, xs):
    xs = list(xs)
    while len(xs) > 1:
        xs = [op(xs[i], xs[i + 1]) for i in range(0, len(xs) - 1, 2)] + (
            [xs[-1]] if len(xs) % 2 else [])
    return xs[0]


@functools.partial(
    pl.kernel,
    compiler_params=pltpu.CompilerParams(needs_layout_passes=False),
    out_type=(
        jax.ShapeDtypeStruct((S,), I32),
        jax.ShapeDtypeStruct((S,), I32),
        jax.ShapeDtypeStruct((S,), I32),
        jax.ShapeDtypeStruct((S,), I32),
        jax.ShapeDtypeStruct((S,), F32),
        jax.ShapeDtypeStruct((S,), F32),
        jax.ShapeDtypeStruct((NW * E,), I32),
        jax.ShapeDtypeStruct((NW * E,), I32),
        jax.ShapeDtypeStruct((NW * E,), F32),
    ),
    mesh=_mesh,
    scratch_types=[
        pltpu.VMEM((E * TPW,), F32),
        pltpu.VMEM((TPW,), I32),
        pltpu.VMEM((TPW,), I32),
        pltpu.VMEM((TPW,), I32),
        pltpu.VMEM((TPW,), I32),
        pltpu.VMEM((TPW,), F32),
        pltpu.VMEM((TPW,), F32),
        pltpu.VMEM((L,), I32),
        pltpu.VMEM((L,), I32),
        pltpu.VMEM((L,), F32),
        pltpu.SemaphoreType.DMA,
    ],
)
def _gate_a(lt_hbm, e1_hbm, e2_hbm, l1_hbm, l2_hbm, g1_hbm, g2_hbm,
            c1_hbm, c2_hbm, gs_hbm,
            lt_v, e1_v, e2_v, l1_v, l2_v, g1_v, g2_v, c1_v, c2_v, gs_v, sem):
    wid = lax.axis_index("s") * NC + lax.axis_index("c")
    base = wid * TPW
    blk = wid // 4
    tok0 = (wid % 4) * TPW
    in_cps = [
        pltpu.async_copy(lt_hbm.at[blk, e, pl.ds(tok0, TPW)],
                         lt_v.at[pl.ds(e * TPW, TPW)], sem)
        for e in range(E)
    ]
    for cp in in_cps:
        cp.wait()

    lanes = lax.iota(I32, L)
    zi = jnp.zeros((L,), I32)
    zf = jnp.zeros((L,), F32)
    neg_inf = jnp.full((L,), -jnp.inf, F32)
    sentinel = jnp.full((L,), E, I32)

    cnt1 = zi
    cnt2 = zi
    acc = [zf] * E
    for g in range(G):
        off = g * L
        le = [lt_v[pl.ds(e * TPW + off, L)] for e in range(E)]
        mx = _tree_reduce(jnp.maximum, le)
        ex = [jnp.exp(le[e] - mx) for e in range(E)]
        ssum = _tree_reduce(jnp.add, ex)
        gates = [ex[e] / ssum for e in range(E)]
        mg = _tree_reduce(jnp.maximum, gates)
        e1 = sentinel
        for e in range(E):
            e1 = jnp.where((gates[e] == mg) & (e1 == E), e, e1)
        g1 = mg
        ml = [jnp.where(e1 == e, neg_inf, le[e]) for e in range(E)]
        m2 = _tree_reduce(jnp.maximum, ml)
        e2 = sentinel
        g2 = zf
        for e in range(E):
            is2 = (ml[e] == m2) & (e2 == E)
            g2 = jnp.where(is2, gates[e], g2)
            e2 = jnp.where(is2, e, e2)
        c1_v[...] = cnt1
        c2_v[...] = cnt2
        prior1 = plsc.load_gather(c1_v, [e1])
        prior2 = plsc.load_gather(c2_v, [e2])
        rank1 = zi
        rank2 = zi
        for e in range(E):
            m1e = e1 == e
            rank1 = jnp.where(m1e, plsc.cumsum(m1e.astype(I32)) - 1, rank1)
            cnt1 = cnt1 + jnp.where(lanes == e,
                                    plsc.all_reduce_population_count(m1e), 0)
            m2e = e2 == e
            rank2 = jnp.where(m2e, plsc.cumsum(m2e.astype(I32)) - 1, rank2)
            cnt2 = cnt2 + jnp.where(lanes == e,
                                    plsc.all_reduce_population_count(m2e), 0)
            acc[e] = acc[e] + gates[e]
        e1_v[pl.ds(off, L)] = e1
        e2_v[pl.ds(off, L)] = e2
        l1_v[pl.ds(off, L)] = prior1 + rank1
        l2_v[pl.ds(off, L)] = prior2 + rank2
        g1_v[pl.ds(off, L)] = g1
        g2_v[pl.ds(off, L)] = g2

    c1_v[...] = cnt1
    c2_v[...] = cnt2
    gsr = zf
    for e in range(E):
        gsr = gsr + jnp.where(lanes == e, jnp.sum(acc[e]), 0.0)
    gs_v[...] = gsr

    cps = [
        pltpu.async_copy(e1_v, e1_hbm.at[pl.ds(base, TPW)], sem),
        pltpu.async_copy(e2_v, e2_hbm.at[pl.ds(base, TPW)], sem),
        pltpu.async_copy(l1_v, l1_hbm.at[pl.ds(base, TPW)], sem),
        pltpu.async_copy(l2_v, l2_hbm.at[pl.ds(base, TPW)], sem),
        pltpu.async_copy(g1_v, g1_hbm.at[pl.ds(base, TPW)], sem),
        pltpu.async_copy(g2_v, g2_hbm.at[pl.ds(base, TPW)], sem),
        pltpu.async_copy(c1_v, c1_hbm.at[pl.ds(wid * E, E)], sem),
        pltpu.async_copy(c2_v, c2_hbm.at[pl.ds(wid * E, E)], sem),
        pltpu.async_copy(gs_v, gs_hbm.at[pl.ds(wid * E, E)], sem),
    ]
    for cp in cps:
        cp.wait()


@functools.partial(
    pl.kernel,
    compiler_params=pltpu.CompilerParams(needs_layout_passes=False),
    out_type=(
        jax.ShapeDtypeStruct((S * C,), F32),
        jax.ShapeDtypeStruct((L,), F32),
    ),
    mesh=_mesh,
    scratch_types=[
        pltpu.VMEM((TPW,), I32),
        pltpu.VMEM((TPW,), I32),
        pltpu.VMEM((TPW,), I32),
        pltpu.VMEM((TPW,), I32),
        pltpu.VMEM((TPW,), F32),
        pltpu.VMEM((TPW,), F32),
        pltpu.VMEM((NW * E,), I32),
        pltpu.VMEM((NW * E,), I32),
        pltpu.VMEM((NW * E,), F32),
        pltpu.VMEM((L,), I32),
        pltpu.VMEM((L,), I32),
        pltpu.VMEM((TPW * C,), F32),
        pltpu.VMEM((L,), F32),
        pltpu.SemaphoreType.DMA,
    ],
)
def _gate_b(e1_hbm, e2_hbm, l1_hbm, l2_hbm, g1_hbm, g2_hbm,
            c1_hbm, c2_hbm, gs_hbm, comb_hbm, laux_hbm,
            e1_v, e2_v, l1_v, l2_v, g1_v, g2_v, c1_v, c2_v, gs_v,
            pre1_v, off2_v, comb_v, laux_v, sem):
    wid = lax.axis_index("s") * NC + lax.axis_index("c")
    base = wid * TPW
    cps = [
        pltpu.async_copy(e1_hbm.at[pl.ds(base, TPW)], e1_v, sem),
        pltpu.async_copy(e2_hbm.at[pl.ds(base, TPW)], e2_v, sem),
        pltpu.async_copy(l1_hbm.at[pl.ds(base, TPW)], l1_v, sem),
        pltpu.async_copy(l2_hbm.at[pl.ds(base, TPW)], l2_v, sem),
        pltpu.async_copy(g1_hbm.at[pl.ds(base, TPW)], g1_v, sem),
        pltpu.async_copy(g2_hbm.at[pl.ds(base, TPW)], g2_v, sem),
        pltpu.async_copy(c1_hbm, c1_v, sem),
        pltpu.async_copy(c2_hbm, c2_v, sem),
        pltpu.async_copy(gs_hbm, gs_v, sem),
    ]
    zf = jnp.zeros((L,), F32)

    def zbody(i, carry):
        for j in range(8):
            comb_v[pl.ds(i * (8 * L) + j * L, L)] = zf
        return carry

    lax.fori_loop(0, TPW * C // (8 * L), zbody, 0)
    for cp in cps:
        cp.wait()

    zi = jnp.zeros((L,), I32)
    pre1 = zi
    pre2 = zi
    tot1 = zi
    gtot = zf
    for j in range(NW):
        r1 = c1_v[pl.ds(j * E, E)]
        r2 = c2_v[pl.ds(j * E, E)]
        gr = gs_v[pl.ds(j * E, E)]
        sel = j < wid
        pre1 = pre1 + jnp.where(sel, r1, zi)
        pre2 = pre2 + jnp.where(sel, r2, zi)
        tot1 = tot1 + r1
        gtot = gtot + gr
    pre1_v[...] = pre1
    off2_v[...] = pre2 + tot1

    lanes = lax.iota(I32, L)
    eps = jnp.float32(jnp.finfo(jnp.float32).eps)
    for g in range(G):
        off = g * L
        e1 = e1_v[pl.ds(off, L)]
        e2 = e2_v[pl.ds(off, L)]
        loc1 = l1_v[pl.ds(off, L)] + plsc.load_gather(pre1_v, [e1])
        loc2 = l2_v[pl.ds(off, L)] + plsc.load_gather(off2_v, [e2])
        k1 = loc1 < C
        k2 = loc2 < C
        g1k = jnp.where(k1, g1_v[pl.ds(off, L)], zf)
        g2k = jnp.where(k2, g2_v[pl.ds(off, L)], zf)
        denom = jnp.maximum(g1k + g2k, eps)
        g1n = g1k / denom
        g2n = g2k / denom
        rowbase = (lanes + off) * C
        plsc.addupdate_scatter(comb_v, [rowbase + loc1], g1n, mask=k1)
        plsc.addupdate_scatter(comb_v, [rowbase + loc2], g2n, mask=k2)

    pltpu.sync_copy(comb_v, comb_hbm.at[pl.ds(base * C, TPW * C)])

    @pl.when(wid == 0)
    def _():
        la = jnp.sum(gtot * tot1.astype(F32)) * (1.0 / (E * S * S))
        laux_v[...] = zf + la
        pltpu.sync_copy(laux_v, laux_hbm)


def kernel(input, W):
    lt = _logits_tc_T(input, W)
    outs_a = _gate_a(lt)
    comb, laux = _gate_b(*outs_a)
    combine_weights = comb.reshape(S, 1, C)
    return laux[0], combine_weights, combine_weights.astype(bool)
